# Initial kernel scaffold; baseline (speedup 1.0000x reference)
import jax
import jax.numpy as jnp
from jax.experimental import pallas as pl


def kernel(x, edge_index, edge_attr, batch, params):
    raise NotImplementedError("write your pallas kernel here")


# speedup vs baseline: 4.1256x; 4.1256x over previous
import functools

import jax
import jax.numpy as jnp
from jax import lax
from jax.experimental import pallas as pl
from jax.experimental.pallas import tpu as pltpu
from jax.experimental.pallas import tpu_sc as plsc

NN = 10000
EE = 64000
NH = 40
NPAD = 10240
NW = 32
EPW = EE // NW
HP = 128
FCH = 128

_SELU_A = 1.6732632423543772
_SELU_S = 1.0507009873554805


def _selu(v):
    return _SELU_S * jnp.where(v > 0, v, _SELU_A * (jnp.exp(v) - 1.0))


def _mm_body(a_ref, w_ref, b_ref, o_ref, *, act):
    acc = jnp.dot(a_ref[...], w_ref[...], preferred_element_type=jnp.float32)
    acc = acc + b_ref[...]
    if act == "selu":
        acc = _selu(acc)
    o_ref[...] = acc


def _mm(a, w, b, act=None, bm=400):
    m, k = a.shape
    nc = w.shape[1]
    return pl.pallas_call(
        functools.partial(_mm_body, act=act),
        grid=(m // bm,),
        in_specs=[
            pl.BlockSpec((bm, k), lambda i: (i, 0)),
            pl.BlockSpec((k, nc), lambda i: (0, 0)),
            pl.BlockSpec((1, nc), lambda i: (0, 0)),
        ],
        out_specs=pl.BlockSpec((bm, nc), lambda i: (i, 0)),
        out_shape=jax.ShapeDtypeStruct((m, nc), jnp.float32),
    )(a, w, b.reshape(1, nc))


def _stats_body(a_ref, o_ref):
    i = pl.program_id(0)
    a = a_ref[...]
    blk = jnp.concatenate(
        [jnp.sum(a, axis=0, keepdims=True), jnp.sum(a * a, axis=0, keepdims=True)],
        axis=0,
    )

    @pl.when(i == 0)
    def _():
        o_ref[...] = blk

    @pl.when(i > 0)
    def _():
        o_ref[...] = o_ref[...] + blk


def _bn_stats(a, bm=400):
    m, k = a.shape
    return pl.pallas_call(
        _stats_body,
        grid=(m // bm,),
        in_specs=[pl.BlockSpec((bm, k), lambda i: (i, 0))],
        out_specs=pl.BlockSpec((2, k), lambda i: (0, 0)),
        out_shape=jax.ShapeDtypeStruct((2, k), jnp.float32),
    )(a)


def _normmm_body(a_ref, st_ref, g_ref, bb_ref, w_ref, b_ref, o_ref):
    inv_n = 1.0 / NN
    mu = st_ref[0:1, :] * inv_n
    var = st_ref[1:2, :] * inv_n - mu * mu
    rstd = lax.rsqrt(var + 1e-5)
    a = _selu(g_ref[...] * (a_ref[...] - mu) * rstd + bb_ref[...])
    o_ref[...] = (
        jnp.dot(a, w_ref[...], preferred_element_type=jnp.float32) + b_ref[...]
    )


def _norm_mm(a, stats, g, bb, w, b, bm=400):
    m, k = a.shape
    nc = w.shape[1]
    return pl.pallas_call(
        _normmm_body,
        grid=(m // bm,),
        in_specs=[
            pl.BlockSpec((bm, k), lambda i: (i, 0)),
            pl.BlockSpec((2, k), lambda i: (0, 0)),
            pl.BlockSpec((1, k), lambda i: (0, 0)),
            pl.BlockSpec((1, k), lambda i: (0, 0)),
            pl.BlockSpec((k, nc), lambda i: (0, 0)),
            pl.BlockSpec((1, nc), lambda i: (0, 0)),
        ],
        out_specs=pl.BlockSpec((bm, nc), lambda i: (i, 0)),
        out_shape=jax.ShapeDtypeStruct((m, nc), jnp.float32),
    )(a, stats, g.reshape(1, k), bb.reshape(1, k), w, b.reshape(1, nc))


def _alpha_body(gxl_ref, gxr_ref, ea_ref, we_ref, attx_ref, o_ref):
    e = jnp.dot(ea_ref[...], we_ref[...], preferred_element_type=jnp.float32)
    m = jnp.maximum(gxl_ref[...] + gxr_ref[...] + e, 0.0)
    alpha = jnp.dot(m, attx_ref[...], preferred_element_type=jnp.float32)
    o_ref[...] = jnp.exp(alpha)


def _alpha(gxl, gxr, ea, we, attx, bm=256):
    e, hc = gxl.shape
    return pl.pallas_call(
        _alpha_body,
        grid=(e // bm,),
        in_specs=[
            pl.BlockSpec((bm, hc), lambda i: (i, 0)),
            pl.BlockSpec((bm, hc), lambda i: (i, 0)),
            pl.BlockSpec((bm, 16), lambda i: (i, 0)),
            pl.BlockSpec((16, hc), lambda i: (0, 0)),
            pl.BlockSpec((hc, HP), lambda i: (0, 0)),
        ],
        out_specs=pl.BlockSpec((bm, HP), lambda i: (i, 0)),
        out_shape=jax.ShapeDtypeStruct((e, HP), jnp.float32),
    )(gxl, gxr, ea, we, attx)


def _wgt_body(gxl_ref, wun_ref, gden_ref, hexp_ref, o_ref):
    w = wun_ref[...] * (1.0 / (gden_ref[...] + 1e-16))
    wf = jnp.dot(w, hexp_ref[...], preferred_element_type=jnp.float32)
    o_ref[...] = (gxl_ref[...] * wf)[None]


def _wgt(gxl, wun, gden, hexp, bm=512):
    e, hc = gxl.shape
    nch = hc // FCH
    return pl.pallas_call(
        _wgt_body,
        grid=(e // bm, nch),
        in_specs=[
            pl.BlockSpec((bm, FCH), lambda i, j: (i, j)),
            pl.BlockSpec((bm, HP), lambda i, j: (i, 0)),
            pl.BlockSpec((bm, HP), lambda i, j: (i, 0)),
            pl.BlockSpec((HP, FCH), lambda i, j: (0, j)),
        ],
        out_specs=pl.BlockSpec((1, bm, FCH), lambda i, j: (j, i, 0)),
        out_shape=jax.ShapeDtypeStruct((nch, e, FCH), jnp.float32),
    )(gxl, wun, gden, hexp)


def _comb_body(p_ref, xres_ref, b_ref, o_ref):
    p = p_ref[...]
    o_ref[...] = _selu(p[0, 0] + p[1, 0] + xres_ref[...] + b_ref[...])


def _combine(parts, xres, b, bm=400):
    m, hc = xres.shape
    nch = hc // FCH
    return pl.pallas_call(
        _comb_body,
        grid=(m // bm, nch),
        in_specs=[
            pl.BlockSpec((2, 1, bm, FCH), lambda i, j: (0, j, i, 0)),
            pl.BlockSpec((bm, FCH), lambda i, j: (i, j)),
            pl.BlockSpec((1, FCH), lambda i, j: (0, j)),
        ],
        out_specs=pl.BlockSpec((bm, FCH), lambda i, j: (i, j)),
        out_shape=jax.ShapeDtypeStruct((m, hc), jnp.float32),
    )(parts, xres, b.reshape(1, hc))


def _add2_body(p_ref, o_ref):
    p = p_ref[...]
    o_ref[...] = p[0] + p[1]


def _add2(p, bm=512):
    _, m, k = p.shape
    return pl.pallas_call(
        _add2_body,
        grid=(m // bm,),
        in_specs=[pl.BlockSpec((2, bm, k), lambda i: (0, i, 0))],
        out_specs=pl.BlockSpec((bm, k), lambda i: (i, 0)),
        out_shape=jax.ShapeDtypeStruct((m, k), jnp.float32),
    )(p)


def _sc_mesh():
    return plsc.VectorSubcoreMesh(core_axis_name="c", subcore_axis_name="s")


def _sc_gather(table, idx3):
    r, d = table.shape
    _, nb, bsz = idx3.shape
    e = NW * nb * bsz

    @functools.partial(
        pl.kernel,
        mesh=_sc_mesh(),
        out_type=jax.ShapeDtypeStruct((e, d), jnp.float32),
        scratch_types=[
            pltpu.VMEM((nb, bsz), jnp.int32),
            pltpu.VMEM((bsz, d), jnp.float32),
            pltpu.SemaphoreType.DMA,
        ],
    )
    def k(table_hbm, idx_hbm, out_hbm, idx_v, rows_v, sem):
        wid = lax.axis_index("s") * 2 + lax.axis_index("c")
        pltpu.sync_copy(idx_hbm.at[wid], idx_v)

        def body(jb, carry):
            pltpu.async_copy(table_hbm.at[idx_v.at[jb]], rows_v, sem).wait()
            base = wid * (nb * bsz) + jb * bsz
            pltpu.sync_copy(rows_v, out_hbm.at[pl.ds(base, bsz)])
            return carry

        lax.fori_loop(0, nb, body, 0)

    return k(table, idx3)


def _sc_scatter(contrib, idx3, zeros_hbm):
    nch, e, f = contrib.shape
    _, nbs, bs = idx3.shape
    rps = NPAD // 16

    @functools.partial(
        pl.kernel,
        mesh=_sc_mesh(),
        out_type=jax.ShapeDtypeStruct((2, nch, NPAD, f), jnp.float32),
        scratch_types=[
            pltpu.VMEM((nbs, bs), jnp.int32),
            pltpu.VMEM((bs, f), jnp.float32),
            pltpu.VMEM_SHARED((NPAD, f), jnp.float32),
        ],
    )
    def k(c_hbm, idx_hbm, z_hbm, out_hbm, idx_v, row_v, slab):
        cid = lax.axis_index("c")
        sid = lax.axis_index("s")
        wid = sid * 2 + cid
        pltpu.sync_copy(idx_hbm.at[wid], idx_v)

        def chunk(j, carry):
            pltpu.sync_copy(z_hbm, slab.at[pl.ds(sid * rps, rps)])
            plsc.subcore_barrier()

            def body(jb, c2):
                pltpu.sync_copy(
                    c_hbm.at[j, pl.ds(wid * EPW + jb * bs, bs)], row_v
                )
                pltpu.sync_copy(row_v, slab.at[idx_v.at[jb]], add=True)
                return c2

            lax.fori_loop(0, nbs, body, 0)
            plsc.subcore_barrier()
            pltpu.sync_copy(
                slab.at[pl.ds(sid * rps, rps)],
                out_hbm.at[cid, j, pl.ds(sid * rps, rps)],
            )
            plsc.subcore_barrier()
            return carry

        lax.fori_loop(0, nch, chunk, 0)

    return k(contrib, idx3, zeros_hbm)


def _gat_layer(h, p, c, src_parts, dst_parts, dst80, ea, zeros128, zeros64):
    hc = NH * c
    hcp = h.shape[1]
    xl = _mm(h, p["Wl"], jnp.zeros((hc,), jnp.float32))
    xr = _mm(h, p["Wr"], jnp.zeros((hc,), jnp.float32))
    xres = _mm(h, p["Wres"], jnp.zeros((hc,), jnp.float32))

    gxl = _sc_gather(xl, src_parts[hc])
    gxr = _sc_gather(xr, dst_parts[hc])

    attx = jnp.zeros((hc, HP), jnp.float32)
    attx = attx.at[jnp.arange(hc), jnp.arange(hc) // c].set(p["att"].reshape(-1))
    wun = _alpha(gxl, gxr, ea, p["We"], attx)

    denp = _sc_scatter(wun.reshape(1, EE, HP), dst80, zeros64)
    den = _add2(denp.reshape(2, NPAD, HP))
    gden = _sc_gather(den, dst_parts[HP])

    hexp = jnp.zeros((HP, hc), jnp.float32)
    hexp = hexp.at[jnp.arange(hc) // c, jnp.arange(hc)].set(1.0)
    contrib = _wgt(gxl, wun, gden, hexp)

    parts = _sc_scatter(contrib, dst80, zeros128)
    return _combine(parts, xres, p["b"])


def kernel(x, edge_index, edge_attr, batch, params):
    src = edge_index[0]
    dst = edge_index[1]

    def part(idx, bsz):
        return idx.reshape(NW, EPW // bsz, bsz)

    src_parts = {640: part(src, 80), 1280: part(src, 80), 2560: part(src, 40)}
    dst_parts = {
        HP: part(dst, 80),
        640: part(dst, 80),
        1280: part(dst, 80),
        2560: part(dst, 40),
    }
    dst80 = part(dst, 80)
    zeros128 = jnp.zeros((NPAD // 16, FCH), jnp.float32)
    zeros64 = jnp.zeros((NPAD // 16, HP), jnp.float32)

    h = x
    for name, c in (("gat1", 16), ("gat2", 32), ("gat3", 64)):
        h = _gat_layer(
            h, params[name], c, src_parts, dst_parts, dst80, edge_attr,
            zeros128, zeros64,
        )

    h1 = _mm(h, params["lin1"]["W"], params["lin1"]["b"])
    st1 = _bn_stats(h1)
    h2 = _norm_mm(
        h1, st1, params["bn1"]["g"], params["bn1"]["b"],
        params["lin2"]["W"], params["lin2"]["b"],
    )
    st2 = _bn_stats(h2)
    w345 = jnp.concatenate(
        [params["lin3"]["W"], params["lin4"]["W"], params["lin5"]["W"]], axis=1
    )
    w345 = jnp.pad(w345, ((0, 0), (0, 128 - 69)))
    b345 = jnp.concatenate(
        [params["lin3"]["b"], params["lin4"]["b"], params["lin5"]["b"]]
    )
    b345 = jnp.pad(b345, (0, 128 - 69))
    hh = _norm_mm(
        h2, st2, params["bn2"]["g"], params["bn2"]["b"], w345, b345
    )
    return (hh[:, :64], hh[:, 64:68], hh[:, 68:69])


# speedup vs baseline: 4.2286x; 1.0250x over previous
import functools

import jax
import jax.numpy as jnp
from jax import lax
from jax.experimental import pallas as pl
from jax.experimental.pallas import tpu as pltpu
from jax.experimental.pallas import tpu_sc as plsc

NN = 10000
EE = 64000
NH = 40
NPAD = 10240
NW = 32
EPW = EE // NW
HP = 128
FCH = 128

_SELU_A = 1.6732632423543772
_SELU_S = 1.0507009873554805


def _selu(v):
    return _SELU_S * jnp.where(v > 0, v, _SELU_A * (jnp.exp(v) - 1.0))


def _mm_body(a_ref, w_ref, b_ref, o_ref, *, act):
    acc = jnp.dot(a_ref[...], w_ref[...], preferred_element_type=jnp.float32)
    acc = acc + b_ref[...]
    if act == "selu":
        acc = _selu(acc)
    o_ref[...] = acc


def _mm(a, w, b, act=None, bm=400):
    m, k = a.shape
    nc = w.shape[1]
    return pl.pallas_call(
        functools.partial(_mm_body, act=act),
        grid=(m // bm,),
        in_specs=[
            pl.BlockSpec((bm, k), lambda i: (i, 0)),
            pl.BlockSpec((k, nc), lambda i: (0, 0)),
            pl.BlockSpec((1, nc), lambda i: (0, 0)),
        ],
        out_specs=pl.BlockSpec((bm, nc), lambda i: (i, 0)),
        out_shape=jax.ShapeDtypeStruct((m, nc), jnp.float32),
    )(a, w, b.reshape(1, nc))


def _stats_body(a_ref, o_ref):
    i = pl.program_id(0)
    a = a_ref[...]
    blk = jnp.concatenate(
        [jnp.sum(a, axis=0, keepdims=True), jnp.sum(a * a, axis=0, keepdims=True)],
        axis=0,
    )

    @pl.when(i == 0)
    def _():
        o_ref[...] = blk

    @pl.when(i > 0)
    def _():
        o_ref[...] = o_ref[...] + blk


def _bn_stats(a, bm=400):
    m, k = a.shape
    return pl.pallas_call(
        _stats_body,
        grid=(m // bm,),
        in_specs=[pl.BlockSpec((bm, k), lambda i: (i, 0))],
        out_specs=pl.BlockSpec((2, k), lambda i: (0, 0)),
        out_shape=jax.ShapeDtypeStruct((2, k), jnp.float32),
    )(a)


def _normmm_body(a_ref, st_ref, g_ref, bb_ref, w_ref, b_ref, o_ref):
    inv_n = 1.0 / NN
    mu = st_ref[0:1, :] * inv_n
    var = st_ref[1:2, :] * inv_n - mu * mu
    rstd = lax.rsqrt(var + 1e-5)
    a = _selu(g_ref[...] * (a_ref[...] - mu) * rstd + bb_ref[...])
    o_ref[...] = (
        jnp.dot(a, w_ref[...], preferred_element_type=jnp.float32) + b_ref[...]
    )


def _norm_mm(a, stats, g, bb, w, b, bm=400):
    m, k = a.shape
    nc = w.shape[1]
    return pl.pallas_call(
        _normmm_body,
        grid=(m // bm,),
        in_specs=[
            pl.BlockSpec((bm, k), lambda i: (i, 0)),
            pl.BlockSpec((2, k), lambda i: (0, 0)),
            pl.BlockSpec((1, k), lambda i: (0, 0)),
            pl.BlockSpec((1, k), lambda i: (0, 0)),
            pl.BlockSpec((k, nc), lambda i: (0, 0)),
            pl.BlockSpec((1, nc), lambda i: (0, 0)),
        ],
        out_specs=pl.BlockSpec((bm, nc), lambda i: (i, 0)),
        out_shape=jax.ShapeDtypeStruct((m, nc), jnp.float32),
    )(a, stats, g.reshape(1, k), bb.reshape(1, k), w, b.reshape(1, nc))


def _alpha_body(gxl_ref, gxr_ref, ea_ref, we_ref, attx_ref, o_ref):
    e = jnp.dot(ea_ref[...], we_ref[...], preferred_element_type=jnp.float32)
    m = jnp.maximum(gxl_ref[...] + gxr_ref[...] + e, 0.0)
    alpha = jnp.dot(m, attx_ref[...], preferred_element_type=jnp.float32)
    o_ref[...] = jnp.exp(alpha)


def _alpha(gxl, gxr, ea, we, attx, bm=256):
    e, hc = gxl.shape
    return pl.pallas_call(
        _alpha_body,
        grid=(e // bm,),
        in_specs=[
            pl.BlockSpec((bm, hc), lambda i: (i, 0)),
            pl.BlockSpec((bm, hc), lambda i: (i, 0)),
            pl.BlockSpec((bm, 16), lambda i: (i, 0)),
            pl.BlockSpec((16, hc), lambda i: (0, 0)),
            pl.BlockSpec((hc, HP), lambda i: (0, 0)),
        ],
        out_specs=pl.BlockSpec((bm, HP), lambda i: (i, 0)),
        out_shape=jax.ShapeDtypeStruct((e, HP), jnp.float32),
    )(gxl, gxr, ea, we, attx)


def _wgt_body(gxl_ref, wun_ref, hexp_ref, o_ref):
    wf = jnp.dot(wun_ref[...], hexp_ref[...], preferred_element_type=jnp.float32)
    o_ref[...] = (gxl_ref[...] * wf)[None]


def _wgt(gxl, wun, hexp, bm=512):
    e, hc = gxl.shape
    nch = hc // FCH
    return pl.pallas_call(
        _wgt_body,
        grid=(e // bm, nch),
        in_specs=[
            pl.BlockSpec((bm, FCH), lambda i, j: (i, j)),
            pl.BlockSpec((bm, HP), lambda i, j: (i, 0)),
            pl.BlockSpec((HP, FCH), lambda i, j: (0, j)),
        ],
        out_specs=pl.BlockSpec((1, bm, FCH), lambda i, j: (j, i, 0)),
        out_shape=jax.ShapeDtypeStruct((nch, e, FCH), jnp.float32),
    )(gxl, wun, hexp)


def _comb_body(p_ref, d_ref, hexp_ref, xres_ref, b_ref, o_ref):
    p = p_ref[...]
    d = d_ref[...]
    den = jnp.dot(
        d[0] + d[1], hexp_ref[...], preferred_element_type=jnp.float32
    )
    num = p[0, 0] + p[1, 0]
    o_ref[...] = _selu(num / (den + 1e-16) + xres_ref[...] + b_ref[...])


def _combine(parts, denp, hexp, xres, b, bm=400):
    m, hc = xres.shape
    nch = hc // FCH
    return pl.pallas_call(
        _comb_body,
        grid=(m // bm, nch),
        in_specs=[
            pl.BlockSpec((2, 1, bm, FCH), lambda i, j: (0, j, i, 0)),
            pl.BlockSpec((2, bm, HP), lambda i, j: (0, i, 0)),
            pl.BlockSpec((HP, FCH), lambda i, j: (0, j)),
            pl.BlockSpec((bm, FCH), lambda i, j: (i, j)),
            pl.BlockSpec((1, FCH), lambda i, j: (0, j)),
        ],
        out_specs=pl.BlockSpec((bm, FCH), lambda i, j: (i, j)),
        out_shape=jax.ShapeDtypeStruct((m, hc), jnp.float32),
    )(parts, denp, hexp, xres, b.reshape(1, hc))


def _sc_mesh():
    return plsc.VectorSubcoreMesh(core_axis_name="c", subcore_axis_name="s")


def _sc_gather(table, idx3):
    r, d = table.shape
    _, nb, bsz = idx3.shape
    e = NW * nb * bsz

    @functools.partial(
        pl.kernel,
        mesh=_sc_mesh(),
        out_type=jax.ShapeDtypeStruct((e, d), jnp.float32),
        scratch_types=[
            pltpu.VMEM((nb, bsz), jnp.int32),
            pltpu.VMEM((bsz, d), jnp.float32),
            pltpu.VMEM((bsz, d), jnp.float32),
            pltpu.SemaphoreType.DMA,
            pltpu.SemaphoreType.DMA,
        ],
    )
    def k(table_hbm, idx_hbm, out_hbm, idx_v, rows_a, rows_b, sem_a, sem_b):
        wid = lax.axis_index("s") * 2 + lax.axis_index("c")
        base0 = wid * (nb * bsz)
        pltpu.sync_copy(idx_hbm.at[wid], idx_v)
        pltpu.async_copy(table_hbm.at[idx_v.at[0]], rows_a, sem_a)

        def body(it, carry):
            jb = it * 2
            pltpu.async_copy(table_hbm.at[idx_v.at[jb + 1]], rows_b, sem_b)
            pltpu.make_async_copy(
                table_hbm.at[idx_v.at[0]], rows_a, sem_a
            ).wait()
            pltpu.sync_copy(rows_a, out_hbm.at[pl.ds(base0 + jb * bsz, bsz)])

            @pl.when(jb + 2 < nb)
            def _():
                pltpu.async_copy(table_hbm.at[idx_v.at[jb + 2]], rows_a, sem_a)

            pltpu.make_async_copy(
                table_hbm.at[idx_v.at[0]], rows_b, sem_b
            ).wait()
            pltpu.sync_copy(
                rows_b, out_hbm.at[pl.ds(base0 + (jb + 1) * bsz, bsz)]
            )
            return carry

        lax.fori_loop(0, nb // 2, body, 0)

    return k(table, idx3)


def _sc_scatter(contrib, idx3, zeros_hbm):
    nch, e, f = contrib.shape
    _, nbs, bs = idx3.shape
    rps = NPAD // 16

    @functools.partial(
        pl.kernel,
        mesh=_sc_mesh(),
        out_type=jax.ShapeDtypeStruct((2, nch, NPAD, f), jnp.float32),
        scratch_types=[
            pltpu.VMEM((nbs, bs), jnp.int32),
            pltpu.VMEM((bs, f), jnp.float32),
            pltpu.VMEM_SHARED((NPAD, f), jnp.float32),
        ],
    )
    def k(c_hbm, idx_hbm, z_hbm, out_hbm, idx_v, row_v, slab):
        cid = lax.axis_index("c")
        sid = lax.axis_index("s")
        wid = sid * 2 + cid
        pltpu.sync_copy(idx_hbm.at[wid], idx_v)

        def chunk(j, carry):
            pltpu.sync_copy(z_hbm, slab.at[pl.ds(sid * rps, rps)])
            plsc.subcore_barrier()

            def body(jb, c2):
                pltpu.sync_copy(
                    c_hbm.at[j, pl.ds(wid * EPW + jb * bs, bs)], row_v
                )
                pltpu.sync_copy(row_v, slab.at[idx_v.at[jb]], add=True)
                return c2

            lax.fori_loop(0, nbs, body, 0)
            plsc.subcore_barrier()
            pltpu.sync_copy(
                slab.at[pl.ds(sid * rps, rps)],
                out_hbm.at[cid, j, pl.ds(sid * rps, rps)],
            )
            plsc.subcore_barrier()
            return carry

        lax.fori_loop(0, nch, chunk, 0)

    return k(contrib, idx3, zeros_hbm)


def _gat_layer(h, p, c, src_parts, dst_parts, dst80, ea, zeros128, zeros64):
    hc = NH * c
    hcp = h.shape[1]
    xl = _mm(h, p["Wl"], jnp.zeros((hc,), jnp.float32))
    xr = _mm(h, p["Wr"], jnp.zeros((hc,), jnp.float32))
    xres = _mm(h, p["Wres"], jnp.zeros((hc,), jnp.float32))

    gxl = _sc_gather(xl, src_parts[hc])
    gxr = _sc_gather(xr, dst_parts[hc])

    attx = jnp.zeros((hc, HP), jnp.float32)
    attx = attx.at[jnp.arange(hc), jnp.arange(hc) // c].set(p["att"].reshape(-1))
    wun = _alpha(gxl, gxr, ea, p["We"], attx)

    denp = _sc_scatter(wun.reshape(1, EE, HP), dst80, zeros64)

    hexp = jnp.zeros((HP, hc), jnp.float32)
    hexp = hexp.at[jnp.arange(hc) // c, jnp.arange(hc)].set(1.0)
    contrib = _wgt(gxl, wun, hexp)

    parts = _sc_scatter(contrib, dst80, zeros128)
    return _combine(parts, denp.reshape(2, NPAD, HP), hexp, xres, p["b"])


def kernel(x, edge_index, edge_attr, batch, params):
    src = edge_index[0]
    dst = edge_index[1]

    def part(idx, bsz):
        return idx.reshape(NW, EPW // bsz, bsz)

    src_parts = {640: part(src, 40), 1280: part(src, 40), 2560: part(src, 8)}
    dst_parts = {
        640: part(dst, 40),
        1280: part(dst, 40),
        2560: part(dst, 8),
    }
    dst80 = part(dst, 80)
    zeros128 = jnp.zeros((NPAD // 16, FCH), jnp.float32)
    zeros64 = jnp.zeros((NPAD // 16, HP), jnp.float32)

    h = x
    for name, c in (("gat1", 16), ("gat2", 32), ("gat3", 64)):
        h = _gat_layer(
            h, params[name], c, src_parts, dst_parts, dst80, edge_attr,
            zeros128, zeros64,
        )

    h1 = _mm(h, params["lin1"]["W"], params["lin1"]["b"])
    st1 = _bn_stats(h1)
    h2 = _norm_mm(
        h1, st1, params["bn1"]["g"], params["bn1"]["b"],
        params["lin2"]["W"], params["lin2"]["b"],
    )
    st2 = _bn_stats(h2)
    w345 = jnp.concatenate(
        [params["lin3"]["W"], params["lin4"]["W"], params["lin5"]["W"]], axis=1
    )
    w345 = jnp.pad(w345, ((0, 0), (0, 128 - 69)))
    b345 = jnp.concatenate(
        [params["lin3"]["b"], params["lin4"]["b"], params["lin5"]["b"]]
    )
    b345 = jnp.pad(b345, (0, 128 - 69))
    hh = _norm_mm(
        h2, st2, params["bn2"]["g"], params["bn2"]["b"], w345, b345
    )
    return (hh[:, :64], hh[:, 64:68], hh[:, 68:69])


# speedup vs baseline: 5.1578x; 1.2197x over previous
import functools

import jax
import jax.numpy as jnp
from jax import lax
from jax.experimental import pallas as pl
from jax.experimental.pallas import tpu as pltpu
from jax.experimental.pallas import tpu_sc as plsc

NN = 10000
EE = 64000
NH = 40
NPAD = 10240
NW = 32
EPW = EE // NW
HP = 128
FCH = 128

_SELU_A = 1.6732632423543772
_SELU_S = 1.0507009873554805


def _selu(v):
    return _SELU_S * jnp.where(v > 0, v, _SELU_A * (jnp.exp(v) - 1.0))


def _mm_body(a_ref, w_ref, b_ref, o_ref, *, act):
    acc = jnp.dot(a_ref[...], w_ref[...], preferred_element_type=jnp.float32)
    acc = acc + b_ref[...]
    if act == "selu":
        acc = _selu(acc)
    o_ref[...] = acc


def _mm(a, w, b, act=None, bm=400):
    m, k = a.shape
    nc = w.shape[1]
    return pl.pallas_call(
        functools.partial(_mm_body, act=act),
        grid=(m // bm,),
        in_specs=[
            pl.BlockSpec((bm, k), lambda i: (i, 0)),
            pl.BlockSpec((k, nc), lambda i: (0, 0)),
            pl.BlockSpec((1, nc), lambda i: (0, 0)),
        ],
        out_specs=pl.BlockSpec((bm, nc), lambda i: (i, 0)),
        out_shape=jax.ShapeDtypeStruct((m, nc), jnp.float32),
    )(a, w, b.reshape(1, nc))


def _stats_body(a_ref, o_ref):
    i = pl.program_id(0)
    a = a_ref[...]
    blk = jnp.concatenate(
        [jnp.sum(a, axis=0, keepdims=True), jnp.sum(a * a, axis=0, keepdims=True)],
        axis=0,
    )

    @pl.when(i == 0)
    def _():
        o_ref[...] = blk

    @pl.when(i > 0)
    def _():
        o_ref[...] = o_ref[...] + blk


def _bn_stats(a, bm=400):
    m, k = a.shape
    return pl.pallas_call(
        _stats_body,
        grid=(m // bm,),
        in_specs=[pl.BlockSpec((bm, k), lambda i: (i, 0))],
        out_specs=pl.BlockSpec((2, k), lambda i: (0, 0)),
        out_shape=jax.ShapeDtypeStruct((2, k), jnp.float32),
    )(a)


def _normmm_body(a_ref, st_ref, g_ref, bb_ref, w_ref, b_ref, o_ref):
    inv_n = 1.0 / NN
    mu = st_ref[0:1, :] * inv_n
    var = st_ref[1:2, :] * inv_n - mu * mu
    rstd = lax.rsqrt(var + 1e-5)
    a = _selu(g_ref[...] * (a_ref[...] - mu) * rstd + bb_ref[...])
    o_ref[...] = (
        jnp.dot(a, w_ref[...], preferred_element_type=jnp.float32) + b_ref[...]
    )


def _norm_mm(a, stats, g, bb, w, b, bm=400):
    m, k = a.shape
    nc = w.shape[1]
    return pl.pallas_call(
        _normmm_body,
        grid=(m // bm,),
        in_specs=[
            pl.BlockSpec((bm, k), lambda i: (i, 0)),
            pl.BlockSpec((2, k), lambda i: (0, 0)),
            pl.BlockSpec((1, k), lambda i: (0, 0)),
            pl.BlockSpec((1, k), lambda i: (0, 0)),
            pl.BlockSpec((k, nc), lambda i: (0, 0)),
            pl.BlockSpec((1, nc), lambda i: (0, 0)),
        ],
        out_specs=pl.BlockSpec((bm, nc), lambda i: (i, 0)),
        out_shape=jax.ShapeDtypeStruct((m, nc), jnp.float32),
    )(a, stats, g.reshape(1, k), bb.reshape(1, k), w, b.reshape(1, nc))


def _alpha_body(gxl_ref, gxr_ref, ea_ref, we_ref, attx_ref, o_ref, o48_ref):
    e = jnp.dot(ea_ref[...], we_ref[...], preferred_element_type=jnp.float32)
    m = jnp.maximum(gxl_ref[...] + gxr_ref[...] + e, 0.0)
    alpha = jnp.dot(m, attx_ref[...], preferred_element_type=jnp.float32)
    w = jnp.exp(alpha)
    o_ref[...] = w
    o48_ref[...] = w[:, :48]


def _alpha(gxl, gxr, ea, we, attx, bm=256):
    e, hc = gxl.shape
    return pl.pallas_call(
        _alpha_body,
        grid=(e // bm,),
        in_specs=[
            pl.BlockSpec((bm, hc), lambda i: (i, 0)),
            pl.BlockSpec((bm, hc), lambda i: (i, 0)),
            pl.BlockSpec((bm, 16), lambda i: (i, 0)),
            pl.BlockSpec((16, hc), lambda i: (0, 0)),
            pl.BlockSpec((hc, HP), lambda i: (0, 0)),
        ],
        out_specs=[
            pl.BlockSpec((bm, HP), lambda i: (i, 0)),
            pl.BlockSpec((bm, 48), lambda i: (i, 0)),
        ],
        out_shape=[
            jax.ShapeDtypeStruct((e, HP), jnp.float32),
            jax.ShapeDtypeStruct((e, 48), jnp.float32),
        ],
    )(gxl, gxr, ea, we, attx)


def _comb_body(p_ref, d_ref, hexp_ref, xres_ref, b_ref, o_ref):
    p = p_ref[...]
    d = d_ref[...]
    den = jnp.dot(
        d[0] + d[1], hexp_ref[...], preferred_element_type=jnp.float32
    )
    num = p[0, 0] + p[1, 0]
    o_ref[...] = _selu(num / (den + 1e-16) + xres_ref[...] + b_ref[...])


def _combine(parts, denp, hexp, xres, b, bm=400):
    m, hc = xres.shape
    nch = hc // FCH
    return pl.pallas_call(
        _comb_body,
        grid=(m // bm, nch),
        in_specs=[
            pl.BlockSpec((2, 1, bm, FCH), lambda i, j: (0, j, i, 0)),
            pl.BlockSpec((2, bm, HP), lambda i, j: (0, i, 0)),
            pl.BlockSpec((HP, FCH), lambda i, j: (0, j)),
            pl.BlockSpec((bm, FCH), lambda i, j: (i, j)),
            pl.BlockSpec((1, FCH), lambda i, j: (0, j)),
        ],
        out_specs=pl.BlockSpec((bm, FCH), lambda i, j: (i, j)),
        out_shape=jax.ShapeDtypeStruct((m, hc), jnp.float32),
    )(parts, denp, hexp, xres, b.reshape(1, hc))


def _sc_mesh():
    return plsc.VectorSubcoreMesh(core_axis_name="c", subcore_axis_name="s")


def _sc_gather(table, idx3):
    r, d = table.shape
    _, nb, bsz = idx3.shape
    e = NW * nb * bsz

    @functools.partial(
        pl.kernel,
        mesh=_sc_mesh(),
        out_type=jax.ShapeDtypeStruct((e, d), jnp.float32),
        scratch_types=[
            pltpu.VMEM((nb, bsz), jnp.int32),
            pltpu.VMEM((bsz, d), jnp.float32),
            pltpu.VMEM((bsz, d), jnp.float32),
            pltpu.SemaphoreType.DMA,
            pltpu.SemaphoreType.DMA,
        ],
    )
    def k(table_hbm, idx_hbm, out_hbm, idx_v, rows_a, rows_b, sem_a, sem_b):
        wid = lax.axis_index("s") * 2 + lax.axis_index("c")
        base0 = wid * (nb * bsz)
        pltpu.sync_copy(idx_hbm.at[wid], idx_v)
        pltpu.async_copy(table_hbm.at[idx_v.at[0]], rows_a, sem_a)

        def body(it, carry):
            jb = it * 2
            pltpu.async_copy(table_hbm.at[idx_v.at[jb + 1]], rows_b, sem_b)
            pltpu.make_async_copy(
                table_hbm.at[idx_v.at[0]], rows_a, sem_a
            ).wait()
            pltpu.sync_copy(rows_a, out_hbm.at[pl.ds(base0 + jb * bsz, bsz)])

            @pl.when(jb + 2 < nb)
            def _():
                pltpu.async_copy(table_hbm.at[idx_v.at[jb + 2]], rows_a, sem_a)

            pltpu.make_async_copy(
                table_hbm.at[idx_v.at[0]], rows_b, sem_b
            ).wait()
            pltpu.sync_copy(
                rows_b, out_hbm.at[pl.ds(base0 + (jb + 1) * bsz, bsz)]
            )
            return carry

        lax.fori_loop(0, nb // 2, body, 0)

    return k(table, idx3)


def _sc_scatter(contrib, idx3, zeros_hbm):
    nch, e, f = contrib.shape
    _, nbs, bs = idx3.shape
    rps = NPAD // 16

    @functools.partial(
        pl.kernel,
        mesh=_sc_mesh(),
        out_type=jax.ShapeDtypeStruct((2, nch, NPAD, f), jnp.float32),
        scratch_types=[
            pltpu.VMEM((nbs, bs), jnp.int32),
            pltpu.VMEM((bs, f), jnp.float32),
            pltpu.VMEM_SHARED((NPAD, f), jnp.float32),
        ],
    )
    def k(c_hbm, idx_hbm, z_hbm, out_hbm, idx_v, row_v, slab):
        cid = lax.axis_index("c")
        sid = lax.axis_index("s")
        wid = sid * 2 + cid
        pltpu.sync_copy(idx_hbm.at[wid], idx_v)

        def chunk(j, carry):
            pltpu.sync_copy(z_hbm, slab.at[pl.ds(sid * rps, rps)])
            plsc.subcore_barrier()

            def body(jb, c2):
                pltpu.sync_copy(
                    c_hbm.at[j, pl.ds(wid * EPW + jb * bs, bs)], row_v
                )
                pltpu.sync_copy(row_v, slab.at[idx_v.at[jb]], add=True)
                return c2

            lax.fori_loop(0, nbs, body, 0)
            plsc.subcore_barrier()
            pltpu.sync_copy(
                slab.at[pl.ds(sid * rps, rps)],
                out_hbm.at[cid, j, pl.ds(sid * rps, rps)],
            )
            plsc.subcore_barrier()
            return carry

        lax.fori_loop(0, nch, chunk, 0)

    return k(contrib, idx3, zeros_hbm)


def _sc_scatter_w(gxl, wun48, idx3, zeros_hbm, c):
    e, hc = gxl.shape
    nch = hc // FCH
    _, nbs, bs = idx3.shape
    rps = NPAD // 16
    hpc = FCH // c
    cpg = c // 16
    zr = zeros_hbm.shape[0]

    @functools.partial(
        pl.kernel,
        mesh=_sc_mesh(),
        out_type=jax.ShapeDtypeStruct((2, nch, NPAD, FCH), jnp.float32),
        scratch_types=[
            pltpu.VMEM((nbs, bs), jnp.int32),
            pltpu.VMEM((bs, FCH), jnp.float32),
            pltpu.VMEM((bs * 48 + 64,), jnp.float32),
            pltpu.VMEM((zr, FCH), jnp.float32),
            pltpu.VMEM_SHARED((NPAD, FCH), jnp.float32),
        ],
    )
    def k(gxl_hbm, wun_hbm, idx_hbm, z_hbm, out_hbm, idx_v, row_v, wun_v,
          zero_v, slab):
        cid = lax.axis_index("c")
        sid = lax.axis_index("s")
        wid = sid * 2 + cid
        pltpu.sync_copy(idx_hbm.at[wid], idx_v)
        pltpu.sync_copy(z_hbm, zero_v)

        def chunk(j, carry):
            for z in range(rps // zr):
                pltpu.sync_copy(
                    zero_v, slab.at[pl.ds(sid * rps + z * zr, zr)]
                )
            plsc.subcore_barrier()

            def body(jb, c2):
                pltpu.sync_copy(
                    gxl_hbm.at[
                        pl.ds(wid * EPW + jb * bs, bs), pl.ds(j * FCH, FCH)
                    ],
                    row_v,
                )
                pltpu.sync_copy(
                    wun_hbm.at[pl.ds((wid * EPW + jb * bs) * 48, bs * 48)],
                    wun_v.at[pl.ds(0, bs * 48)],
                )

                def edge(i, c3):
                    wv = wun_v[pl.ds(i * 48 + j * hpc, 16)]
                    for g in range(FCH // 16):
                        w = wv[g // cpg]
                        row_v[i, pl.ds(g * 16, 16)] = (
                            w * row_v[i, pl.ds(g * 16, 16)]
                        )
                    return c3

                lax.fori_loop(0, bs, edge, 0)
                pltpu.sync_copy(row_v, slab.at[idx_v.at[jb]], add=True)
                return c2

            lax.fori_loop(0, nbs, body, 0)
            plsc.subcore_barrier()
            pltpu.sync_copy(
                slab.at[pl.ds(sid * rps, rps)],
                out_hbm.at[cid, j, pl.ds(sid * rps, rps)],
            )
            plsc.subcore_barrier()
            return carry

        lax.fori_loop(0, nch, chunk, 0)

    return k(gxl, wun48, idx3, zeros_hbm)


def _gat_layer(h, p, c, src_parts, dst_parts, dst80, ea, zeros128, zeros64):
    hc = NH * c
    hcp = h.shape[1]
    xl = _mm(h, p["Wl"], jnp.zeros((hc,), jnp.float32))
    xr = _mm(h, p["Wr"], jnp.zeros((hc,), jnp.float32))
    xres = _mm(h, p["Wres"], jnp.zeros((hc,), jnp.float32))

    gxl = _sc_gather(xl, src_parts[hc])
    gxr = _sc_gather(xr, dst_parts[hc])

    attx = jnp.zeros((hc, HP), jnp.float32)
    attx = attx.at[jnp.arange(hc), jnp.arange(hc) // c].set(p["att"].reshape(-1))
    wun, wun48 = _alpha(gxl, gxr, ea, p["We"], attx)

    denp = _sc_scatter(wun.reshape(1, EE, HP), dst80, zeros64)

    hexp = jnp.zeros((HP, hc), jnp.float32)
    hexp = hexp.at[jnp.arange(hc) // c, jnp.arange(hc)].set(1.0)
    parts = _sc_scatter_w(gxl, wun48.reshape(-1), dst80, zeros128, c)
    return _combine(parts, denp.reshape(2, NPAD, HP), hexp, xres, p["b"])


def kernel(x, edge_index, edge_attr, batch, params):
    src = edge_index[0]
    dst = edge_index[1]

    def part(idx, bsz):
        return idx.reshape(NW, EPW // bsz, bsz)

    src_parts = {640: part(src, 40), 1280: part(src, 40), 2560: part(src, 8)}
    dst_parts = {
        640: part(dst, 40),
        1280: part(dst, 40),
        2560: part(dst, 8),
    }
    dst80 = part(dst, 80)
    zeros128 = jnp.zeros((128, FCH), jnp.float32)
    zeros64 = jnp.zeros((NPAD // 16, HP), jnp.float32)

    h = x
    for name, c in (("gat1", 16), ("gat2", 32), ("gat3", 64)):
        h = _gat_layer(
            h, params[name], c, src_parts, dst_parts, dst80, edge_attr,
            zeros128, zeros64,
        )

    h1 = _mm(h, params["lin1"]["W"], params["lin1"]["b"])
    st1 = _bn_stats(h1)
    h2 = _norm_mm(
        h1, st1, params["bn1"]["g"], params["bn1"]["b"],
        params["lin2"]["W"], params["lin2"]["b"],
    )
    st2 = _bn_stats(h2)
    w345 = jnp.concatenate(
        [params["lin3"]["W"], params["lin4"]["W"], params["lin5"]["W"]], axis=1
    )
    w345 = jnp.pad(w345, ((0, 0), (0, 128 - 69)))
    b345 = jnp.concatenate(
        [params["lin3"]["b"], params["lin4"]["b"], params["lin5"]["b"]]
    )
    b345 = jnp.pad(b345, (0, 128 - 69))
    hh = _norm_mm(
        h2, st2, params["bn2"]["g"], params["bn2"]["b"], w345, b345
    )
    return (hh[:, :64], hh[:, 64:68], hh[:, 68:69])
